# baseline (speedup 1.0000x reference)
import functools

import jax
import jax.numpy as jnp
from jax import lax
from jax.experimental import pallas as pl
from jax.experimental.pallas import tpu as pltpu
from jax.experimental.pallas import tpu_sc as plsc

_V = 63
_D = 64
_VPAD = 64
_IDX_PER_STREAM = 128


def _table_body(sym_ref, w_ref, b_ref, out_ref):
    out_ref[...] = (
        jnp.dot(sym_ref[...], w_ref[...], preferred_element_type=jnp.float32)
        + b_ref[0:1, :]
    )


def _fused_table(symbols, W, b):
    sym = jnp.pad(symbols, ((0, _VPAD - symbols.shape[0]), (0, 0)))
    b2 = jnp.broadcast_to(b.reshape(1, -1), (8, _D))
    return pl.pallas_call(
        _table_body,
        out_shape=jax.ShapeDtypeStruct((_VPAD, _D), jnp.float32),
    )(sym, W, b2)


@functools.cache
def _make_gather(tot, d):
    info = plsc.get_sparse_core_info()
    nc, ns = info.num_cores, info.num_subcores
    nw = nc * ns
    per_w = tot // nw
    ch = 640
    ksub = ch // _IDX_PER_STREAM
    nch = per_w // ch
    nit = nch // 2
    rows_per_w = per_w // _IDX_PER_STREAM
    mesh = plsc.VectorSubcoreMesh(core_axis_name="c", subcore_axis_name="s")

    @functools.partial(
        pl.kernel,
        out_type=jax.ShapeDtypeStruct((tot, d), jnp.float32),
        mesh=mesh,
        scratch_types=[
            pltpu.VMEM((ksub, _IDX_PER_STREAM), jnp.int32),
            pltpu.VMEM((ksub, _IDX_PER_STREAM), jnp.int32),
            pltpu.VMEM((ch, d), jnp.float32),
            pltpu.VMEM((ch, d), jnp.float32),
            pltpu.SemaphoreType.DMA,
            pltpu.SemaphoreType.DMA,
            pltpu.SemaphoreType.DMA,
            pltpu.SemaphoreType.DMA,
        ],
        compiler_params=pltpu.CompilerParams(use_tc_tiling_on_sc=False),
    )
    def gather(table_hbm, qr_hbm, out_hbm, idx0, idx1, rows0, rows1,
               gs0, gs1, os0, os1):
        wid = lax.axis_index("s") * nc + lax.axis_index("c")
        base = wid * per_w
        row_base = wid * rows_per_w
        bufs = ((idx0, rows0, gs0, os0), (idx1, rows1, gs1, os1))

        pltpu.sync_copy(qr_hbm.at[pl.ds(row_base, ksub)], idx0)
        pltpu.sync_copy(qr_hbm.at[pl.ds(row_base + ksub, ksub)], idx1)

        def body(k, carry):
            for s, (idxb, rowsb, gsem, osem) in enumerate(bufs):
                g = 2 * k + s

                @pl.when(k > 0)
                def _():
                    pltpu.make_async_copy(
                        rowsb, out_hbm.at[pl.ds(base, ch)], osem).wait()

                hs = [
                    pltpu.async_copy(
                        table_hbm.at[idxb.at[j]],
                        rowsb.at[pl.ds(j * _IDX_PER_STREAM, _IDX_PER_STREAM)],
                        gsem,
                    )
                    for j in range(ksub)
                ]
                for h in hs:
                    h.wait()
                pltpu.async_copy(rowsb, out_hbm.at[pl.ds(base + g * ch, ch)],
                                 osem)

                @pl.when(g + 2 < nch)
                def _():
                    pltpu.sync_copy(
                        qr_hbm.at[pl.ds(row_base + (g + 2) * ksub, ksub)],
                        idxb)
            return carry

        lax.fori_loop(0, nit, body, 0)

        for _, rowsb, _, osem in bufs:
            pltpu.make_async_copy(rowsb, out_hbm.at[pl.ds(base, ch)],
                                  osem).wait()

    return gather


def kernel(QR, symbols, W, b):
    bsz, seq = QR.shape
    tot = bsz * seq
    table = _fused_table(symbols, W, b)
    qr2 = QR.reshape(tot // _IDX_PER_STREAM, _IDX_PER_STREAM).astype(jnp.int32)
    out = _make_gather(tot, _D)(table, qr2)
    return out.reshape(bsz, seq, _D)


# speedup vs baseline: 1.0019x; 1.0019x over previous
import functools

import jax
import jax.numpy as jnp
from jax import lax
from jax.experimental import pallas as pl
from jax.experimental.pallas import tpu as pltpu
from jax.experimental.pallas import tpu_sc as plsc

_V = 63
_D = 64
_VPAD = 64
_IDX_PER_STREAM = 128


def _table_body(sym_ref, w_ref, b_ref, out_ref):
    out_ref[...] = (
        jnp.dot(sym_ref[...], w_ref[...], preferred_element_type=jnp.float32)
        + b_ref[0:1, :]
    )


def _fused_table(symbols, W, b):
    sym = jnp.pad(symbols, ((0, _VPAD - symbols.shape[0]), (0, 0)))
    b2 = jnp.broadcast_to(b.reshape(1, -1), (8, _D))
    return pl.pallas_call(
        _table_body,
        out_shape=jax.ShapeDtypeStruct((_VPAD, _D), jnp.float32),
    )(sym, W, b2)


@functools.cache
def _make_gather(bsz, seq, d):
    info = plsc.get_sparse_core_info()
    nc, ns = info.num_cores, info.num_subcores
    nw = nc * ns
    rows_w = bsz // nw
    cb = 4
    ch = cb * seq
    nch = rows_w // cb
    nit = nch // 2
    splits = []
    off = 0
    while off < ch:
        n = min(_IDX_PER_STREAM, ch - off)
        splits.append((off, n))
        off += n
    mesh = plsc.VectorSubcoreMesh(core_axis_name="c", subcore_axis_name="s")

    @functools.partial(
        pl.kernel,
        out_type=jax.ShapeDtypeStruct((bsz, seq, d), jnp.float32),
        mesh=mesh,
        scratch_types=[
            pltpu.VMEM((ch,), jnp.int32),
            pltpu.VMEM((ch,), jnp.int32),
            pltpu.VMEM((ch, d), jnp.float32),
            pltpu.VMEM((ch, d), jnp.float32),
            pltpu.SemaphoreType.DMA,
            pltpu.SemaphoreType.DMA,
            pltpu.SemaphoreType.DMA,
            pltpu.SemaphoreType.DMA,
        ],
        compiler_params=pltpu.CompilerParams(use_tc_tiling_on_sc=False),
    )
    def gather(table_hbm, qr_hbm, out_hbm, idx0, idx1, rows0, rows1,
               gs0, gs1, os0, os1):
        wid = lax.axis_index("s") * nc + lax.axis_index("c")
        bbase = wid * rows_w
        bufs = ((idx0, rows0, gs0, os0), (idx1, rows1, gs1, os1))

        pltpu.sync_copy(qr_hbm.at[pl.ds(bbase * seq, ch)], idx0)
        pltpu.sync_copy(qr_hbm.at[pl.ds(bbase * seq + ch, ch)], idx1)

        def body(k, carry):
            for s, (idxb, rowsb, gsem, osem) in enumerate(bufs):
                g = 2 * k + s
                b0 = bbase + g * cb

                @pl.when(k > 0)
                def _():
                    for _r in range(cb):
                        pltpu.make_async_copy(
                            rowsb.at[pl.ds(0, seq)], out_hbm.at[bbase],
                            osem).wait()

                hs = [
                    pltpu.async_copy(
                        table_hbm.at[idxb.at[pl.ds(o, n)]],
                        rowsb.at[pl.ds(o, n)],
                        gsem,
                    )
                    for o, n in splits
                ]
                for h in hs:
                    h.wait()
                for r in range(cb):
                    pltpu.async_copy(rowsb.at[pl.ds(r * seq, seq)],
                                     out_hbm.at[b0 + r], osem)

                @pl.when(g + 2 < nch)
                def _():
                    pltpu.sync_copy(
                        qr_hbm.at[pl.ds((bbase + (g + 2) * cb) * seq, ch)],
                        idxb)
            return carry

        lax.fori_loop(0, nit, body, 0)

        for _, rowsb, _, osem in bufs:
            for _r in range(cb):
                pltpu.make_async_copy(rowsb.at[pl.ds(0, seq)],
                                      out_hbm.at[bbase], osem).wait()

    return gather


def kernel(QR, symbols, W, b):
    bsz, seq = QR.shape
    table = _fused_table(symbols, W, b)
    qr1 = QR.reshape(-1).astype(jnp.int32)
    return _make_gather(bsz, seq, _D)(table, qr1)


# speedup vs baseline: 2.6364x; 2.6313x over previous
import functools

import jax
import jax.numpy as jnp
from jax import lax
from jax.experimental import pallas as pl
from jax.experimental.pallas import tpu as pltpu
from jax.experimental.pallas import tpu_sc as plsc

_V = 63
_D = 64
_VPAD = 64
_IDX_PER_STREAM = 128


def _table_body(sym_ref, w_ref, b_ref, out_ref):
    out_ref[...] = (
        jnp.dot(sym_ref[...], w_ref[...], preferred_element_type=jnp.float32)
        + b_ref[0:1, :]
    )


def _fused_table(symbols, W, b):
    sym = jnp.pad(symbols, ((0, _VPAD - symbols.shape[0]), (0, 0)))
    b2 = jnp.broadcast_to(b.reshape(1, -1), (8, _D))
    return pl.pallas_call(
        _table_body,
        out_shape=jax.ShapeDtypeStruct((_VPAD, _D), jnp.float32),
    )(sym, W, b2)


@functools.cache
def _make_gather(bsz, seq, d):
    info = plsc.get_sparse_core_info()
    nc, ns = info.num_cores, info.num_subcores
    nw = nc * ns
    rows_w = bsz // nw
    cb = 4
    ch = cb * seq
    nch = rows_w // cb
    nit = nch // 2
    splits = []
    off = 0
    while off < ch:
        n = min(_IDX_PER_STREAM, ch - off)
        splits.append((off, n))
        off += n
    mesh = plsc.VectorSubcoreMesh(core_axis_name="c", subcore_axis_name="s")

    @functools.partial(
        pl.kernel,
        out_type=jax.ShapeDtypeStruct((bsz, seq, d), jnp.float32),
        mesh=mesh,
        scratch_types=[
            pltpu.VMEM((ch,), jnp.int32),
            pltpu.VMEM((ch,), jnp.int32),
            pltpu.VMEM((ch, d), jnp.float32),
            pltpu.VMEM((ch, d), jnp.float32),
            pltpu.VMEM_SHARED((_VPAD, d), jnp.float32),
            pltpu.SemaphoreType.DMA,
            pltpu.SemaphoreType.DMA,
            pltpu.SemaphoreType.DMA,
            pltpu.SemaphoreType.DMA,
        ],
        compiler_params=pltpu.CompilerParams(use_tc_tiling_on_sc=False),
    )
    def gather(table_hbm, qr_hbm, out_hbm, idx0, idx1, rows0, rows1,
               table_sh, gs0, gs1, os0, os1):
        wid = lax.axis_index("s") * nc + lax.axis_index("c")
        bbase = wid * rows_w
        bufs = ((idx0, rows0, gs0, os0), (idx1, rows1, gs1, os1))

        @pl.when(lax.axis_index("s") == 0)
        def _():
            pltpu.sync_copy(table_hbm, table_sh)
        plsc.subcore_barrier()

        pltpu.sync_copy(qr_hbm.at[pl.ds(bbase * seq, ch)], idx0)
        pltpu.sync_copy(qr_hbm.at[pl.ds(bbase * seq + ch, ch)], idx1)

        def body(k, carry):
            for s, (idxb, rowsb, gsem, osem) in enumerate(bufs):
                g = 2 * k + s
                b0 = bbase + g * cb

                @pl.when(k > 0)
                def _():
                    for _r in range(cb):
                        pltpu.make_async_copy(
                            rowsb.at[pl.ds(0, seq)], out_hbm.at[bbase],
                            osem).wait()

                hs = [
                    pltpu.async_copy(
                        table_sh.at[idxb.at[pl.ds(o, n)]],
                        rowsb.at[pl.ds(o, n)],
                        gsem,
                    )
                    for o, n in splits
                ]
                for h in hs:
                    h.wait()
                for r in range(cb):
                    pltpu.async_copy(rowsb.at[pl.ds(r * seq, seq)],
                                     out_hbm.at[b0 + r], osem)

                @pl.when(g + 2 < nch)
                def _():
                    pltpu.sync_copy(
                        qr_hbm.at[pl.ds((bbase + (g + 2) * cb) * seq, ch)],
                        idxb)
            return carry

        lax.fori_loop(0, nit, body, 0)

        for _, rowsb, _, osem in bufs:
            for _r in range(cb):
                pltpu.make_async_copy(rowsb.at[pl.ds(0, seq)],
                                      out_hbm.at[bbase], osem).wait()

    return gather


def kernel(QR, symbols, W, b):
    bsz, seq = QR.shape
    table = _fused_table(symbols, W, b)
    qr1 = QR.reshape(-1).astype(jnp.int32)
    return _make_gather(bsz, seq, _D)(table, qr1)
